# Initial kernel scaffold; baseline (speedup 1.0000x reference)
import jax
import jax.numpy as jnp
from jax.experimental import pallas as pl


def kernel(x, edge_index, edge_attr, W1, b1, W2, b2):
    raise NotImplementedError("write your pallas kernel here")


# speedup vs baseline: 3.2493x; 3.2493x over previous
import jax
import jax.numpy as jnp
from jax import lax
from jax.experimental import pallas as pl
from jax.experimental.pallas import tpu as pltpu
from jax.experimental.pallas import tpu_sc as plsc

_NC = 2
_NS = 16
_L = 16
_C = 80


def _sc_agg(x, src, dst, edge_attr, zrows):
    N, D = x.shape
    E = src.shape[0]
    assert E % (_C * _NC * _NS) == 0 and D % _L == 0
    kpt = E // (_C * _NC * _NS)
    npad = -(-N // (8 * _NS)) * (8 * _NS)
    rps = npad // _NS

    def body(x_hbm, src_hbm, dst_hbm, attr_hbm, z_hbm, out_hbm,
             src_idx, dst_idx, rows, attr, msg, acc, gsem):
        cid = lax.axis_index("c")
        sid = lax.axis_index("s")
        wid = sid * _NC + cid

        r0 = sid * rps
        pltpu.sync_copy(z_hbm, acc.at[pl.ds(r0, rps), :])
        plsc.subcore_barrier()

        def step(k, _):
            base = (wid * kpt + k) * _C
            pltpu.sync_copy(src_hbm.at[pl.ds(base, _C)], src_idx)
            pltpu.sync_copy(dst_hbm.at[pl.ds(base, _C)], dst_idx)
            pltpu.sync_copy(attr_hbm.at[pl.ds(base, _C), :], attr)
            pltpu.async_copy(x_hbm.at[src_idx], rows, gsem).wait()

            def compute(e, _):
                for j in range(D // _L):
                    s = pl.ds(j * _L, _L)
                    msg[e, s] = jnp.maximum(rows[e, s] + attr[e, s], 0.0)
                return 0
            lax.fori_loop(0, _C, compute, 0)
            pltpu.sync_copy(msg, acc.at[dst_idx], add=True)
            return 0
        lax.fori_loop(0, kpt, step, 0)

        plsc.subcore_barrier()
        pltpu.sync_copy(acc.at[pl.ds(r0, rps), :],
                        out_hbm.at[cid, pl.ds(r0, rps), :])

    mesh = plsc.VectorSubcoreMesh(core_axis_name="c", subcore_axis_name="s")
    return pl.kernel(
        body,
        out_type=jax.ShapeDtypeStruct((_NC, npad, D), jnp.float32),
        mesh=mesh,
        scratch_types=[
            pltpu.VMEM((_C,), jnp.int32),
            pltpu.VMEM((_C,), jnp.int32),
            pltpu.VMEM((_C, D), jnp.float32),
            pltpu.VMEM((_C, D), jnp.float32),
            pltpu.VMEM((_C, D), jnp.float32),
            pltpu.VMEM_SHARED((npad, D), jnp.float32),
            pltpu.SemaphoreType.DMA,
        ],
    )(x, src, dst, edge_attr, zrows)


def _mlp_body(x_ref, parts_ref, w1_ref, b1_ref, w2_ref, b2_ref, o_ref):
    h = x_ref[...] + parts_ref[0] + parts_ref[1]
    t = jnp.dot(h, w1_ref[...], preferred_element_type=jnp.float32,
                precision=jax.lax.Precision.HIGHEST) + b1_ref[...]
    t = jnp.maximum(t, 0.0)
    o_ref[...] = jnp.dot(t, w2_ref[...], preferred_element_type=jnp.float32,
                         precision=jax.lax.Precision.HIGHEST) + b2_ref[...]


def _tc_mlp(x, parts, W1, b1, W2, b2):
    N, D = x.shape
    R = 1000
    assert N % R == 0
    return pl.pallas_call(
        _mlp_body,
        grid=(N // R,),
        in_specs=[
            pl.BlockSpec((R, D), lambda i: (i, 0)),
            pl.BlockSpec((_NC, R, D), lambda i: (0, i, 0)),
            pl.BlockSpec((D, D), lambda i: (0, 0)),
            pl.BlockSpec((1, D), lambda i: (0, 0)),
            pl.BlockSpec((D, D), lambda i: (0, 0)),
            pl.BlockSpec((1, D), lambda i: (0, 0)),
        ],
        out_specs=pl.BlockSpec((R, D), lambda i: (i, 0)),
        out_shape=jax.ShapeDtypeStruct((N, D), jnp.float32),
    )(x, parts, W1, b1.reshape(1, D), W2, b2.reshape(1, D))


def kernel(x, edge_index, edge_attr, W1, b1, W2, b2):
    src = edge_index[0]
    dst = edge_index[1]
    N, D = x.shape
    npad = -(-N // (8 * _NS)) * (8 * _NS)
    zrows = jnp.zeros((npad // _NS, D), jnp.float32)
    parts = _sc_agg(x, src, dst, edge_attr, zrows)
    return _tc_mlp(x, parts, W1, b1, W2, b2)


# speedup vs baseline: 4.3078x; 1.3258x over previous
import jax
import jax.numpy as jnp
from jax import lax
from jax.experimental import pallas as pl
from jax.experimental.pallas import tpu as pltpu
from jax.experimental.pallas import tpu_sc as plsc

_NC = 2
_NS = 16
_L = 16
_C = 40


def _sc_agg(x, src, dst, edge_attr, zrows):
    N, D = x.shape
    E = src.shape[0]
    assert E % (_C * _NC * _NS) == 0 and D % _L == 0
    kpt = E // (_C * _NC * _NS)
    assert kpt % 2 == 0
    npad = -(-N // (8 * _NS)) * (8 * _NS)
    rps = npad // _NS

    def body(x_hbm, src_hbm, dst_hbm, attr_hbm, z_hbm, out_hbm,
             src0, src1, dst0, dst1, rows0, rows1, attr0, attr1, msg, acc,
             l0, l1, g0, g1):
        cid = lax.axis_index("c")
        sid = lax.axis_index("s")
        wid = sid * _NC + cid
        srcs, dsts = (src0, src1), (dst0, dst1)
        rowss, attrs = (rows0, rows1), (attr0, attr1)
        lsems, gsems = (l0, l1), (g0, g1)

        r0 = sid * rps
        pltpu.sync_copy(z_hbm, acc.at[pl.ds(r0, rps), :])
        plsc.subcore_barrier()

        def loads(b, k, issue):
            base = (wid * kpt + k) * _C
            trios = ((src_hbm.at[pl.ds(base, _C)], srcs[b]),
                     (dst_hbm.at[pl.ds(base, _C)], dsts[b]),
                     (attr_hbm.at[pl.ds(base, _C), :], attrs[b]))
            for s_ref, d_ref in trios:
                if issue:
                    pltpu.async_copy(s_ref, d_ref, lsems[b])
                else:
                    pltpu.make_async_copy(s_ref, d_ref, lsems[b]).wait()

        def gather(b, issue):
            if issue:
                pltpu.async_copy(x_hbm.at[srcs[b]], rowss[b], gsems[b])
            else:
                pltpu.make_async_copy(x_hbm.at[srcs[b]], rowss[b], gsems[b]).wait()

        loads(0, 0, True)
        loads(1, 1, True)
        loads(0, 0, False)
        gather(0, True)

        def outer(g, _):
            for b in range(2):
                k = 2 * g + b
                nb = 1 - b
                gather(b, False)

                @pl.when(k + 1 < kpt)
                def _():
                    loads(nb, k + 1, False)
                    gather(nb, True)

                @plsc.parallel_loop(0, _C, unroll=2)
                def _(e):
                    for j in range(D // _L):
                        s = pl.ds(j * _L, _L)
                        msg[e, s] = jnp.maximum(rowss[b][e, s] + attrs[b][e, s],
                                                0.0)

                pltpu.sync_copy(msg, acc.at[dsts[b]], add=True)

                @pl.when(k + 2 < kpt)
                def _():
                    loads(b, k + 2, True)
            return 0
        lax.fori_loop(0, kpt // 2, outer, 0)

        plsc.subcore_barrier()
        pltpu.sync_copy(acc.at[pl.ds(r0, rps), :],
                        out_hbm.at[cid, pl.ds(r0, rps), :])

    mesh = plsc.VectorSubcoreMesh(core_axis_name="c", subcore_axis_name="s")
    return pl.kernel(
        body,
        out_type=jax.ShapeDtypeStruct((_NC, npad, D), jnp.float32),
        mesh=mesh,
        scratch_types=[
            pltpu.VMEM((_C,), jnp.int32),
            pltpu.VMEM((_C,), jnp.int32),
            pltpu.VMEM((_C,), jnp.int32),
            pltpu.VMEM((_C,), jnp.int32),
            pltpu.VMEM((_C, D), jnp.float32),
            pltpu.VMEM((_C, D), jnp.float32),
            pltpu.VMEM((_C, D), jnp.float32),
            pltpu.VMEM((_C, D), jnp.float32),
            pltpu.VMEM((_C, D), jnp.float32),
            pltpu.VMEM_SHARED((npad, D), jnp.float32),
            pltpu.SemaphoreType.DMA,
            pltpu.SemaphoreType.DMA,
            pltpu.SemaphoreType.DMA,
            pltpu.SemaphoreType.DMA,
        ],
    )(x, src, dst, edge_attr, zrows)


def _mlp_body(x_ref, parts_ref, w1_ref, b1_ref, w2_ref, b2_ref, o_ref):
    h = x_ref[...] + parts_ref[0] + parts_ref[1]
    t = jnp.dot(h, w1_ref[...], preferred_element_type=jnp.float32,
                precision=jax.lax.Precision.HIGHEST) + b1_ref[...]
    t = jnp.maximum(t, 0.0)
    o_ref[...] = jnp.dot(t, w2_ref[...], preferred_element_type=jnp.float32,
                         precision=jax.lax.Precision.HIGHEST) + b2_ref[...]


def _tc_mlp(x, parts, W1, b1, W2, b2):
    N, D = x.shape
    R = 1000
    assert N % R == 0
    return pl.pallas_call(
        _mlp_body,
        grid=(N // R,),
        in_specs=[
            pl.BlockSpec((R, D), lambda i: (i, 0)),
            pl.BlockSpec((_NC, R, D), lambda i: (0, i, 0)),
            pl.BlockSpec((D, D), lambda i: (0, 0)),
            pl.BlockSpec((1, D), lambda i: (0, 0)),
            pl.BlockSpec((D, D), lambda i: (0, 0)),
            pl.BlockSpec((1, D), lambda i: (0, 0)),
        ],
        out_specs=pl.BlockSpec((R, D), lambda i: (i, 0)),
        out_shape=jax.ShapeDtypeStruct((N, D), jnp.float32),
    )(x, parts, W1, b1.reshape(1, D), W2, b2.reshape(1, D))


def kernel(x, edge_index, edge_attr, W1, b1, W2, b2):
    src = edge_index[0]
    dst = edge_index[1]
    N, D = x.shape
    npad = -(-N // (8 * _NS)) * (8 * _NS)
    zrows = jnp.zeros((npad // _NS, D), jnp.float32)
    parts = _sc_agg(x, src, dst, edge_attr, zrows)
    return _tc_mlp(x, parts, W1, b1, W2, b2)


# speedup vs baseline: 4.9240x; 1.1430x over previous
import jax
import jax.numpy as jnp
from jax import lax
from jax.experimental import pallas as pl
from jax.experimental.pallas import tpu as pltpu
from jax.experimental.pallas import tpu_sc as plsc

_NC = 2
_NS = 16
_L = 16
_C = 40


def _sc_agg(x, src, dst, edge_attr, zrows):
    N, D = x.shape
    E = src.shape[0]
    assert E % (_C * _NC * _NS) == 0 and D % _L == 0
    kpt = E // (_C * _NC * _NS)
    assert kpt % 2 == 0
    npad = -(-N // (8 * _NS)) * (8 * _NS)
    rps = npad // _NS

    def body(x_hbm, src_hbm, dst_hbm, attr_hbm, z_hbm, out_hbm,
             src0, src1, dst0, dst1, dst2, dst3, rows0, rows1,
             attr0, attr1, msg0, msg1, acc,
             l0, l1, g0, g1, s0, s1, d0, d1, d2, d3):
        cid = lax.axis_index("c")
        sid = lax.axis_index("s")
        wid = sid * _NC + cid
        srcs, dsts = (src0, src1), (dst0, dst1, dst2, dst3)
        rowss, attrs = (rows0, rows1), (attr0, attr1)
        msgs = (msg0, msg1)
        lsems, gsems, ssems = (l0, l1), (g0, g1), (s0, s1)
        dsems = (d0, d1, d2, d3)

        r0 = sid * rps
        pltpu.sync_copy(z_hbm, acc.at[pl.ds(r0, rps), :])
        plsc.subcore_barrier()

        def loads(b, ds, k, issue):
            base = (wid * kpt + k) * _C
            pairs = ((src_hbm.at[pl.ds(base, _C)], srcs[b], lsems[b]),
                     (attr_hbm.at[pl.ds(base, _C), :], attrs[b], lsems[b]),
                     (dst_hbm.at[pl.ds(base, _C)], dsts[ds], dsems[ds]))
            for s_ref, d_ref, sem in pairs:
                if issue:
                    pltpu.async_copy(s_ref, d_ref, sem)
                else:
                    pltpu.make_async_copy(s_ref, d_ref, sem).wait()

        def gather(b, issue):
            if issue:
                pltpu.async_copy(x_hbm.at[srcs[b]], rowss[b], gsems[b])
            else:
                pltpu.make_async_copy(x_hbm.at[srcs[b]], rowss[b], gsems[b]).wait()

        def scat(b, ds, issue):
            if issue:
                pltpu.async_copy(msgs[b], acc.at[dsts[ds]], ssems[b], add=True)
            else:
                pltpu.make_async_copy(msgs[b], acc.at[dsts[ds]], ssems[b]).wait()

        loads(0, 0, 0, True)
        loads(1, 1, 1, True)
        loads(0, 0, 0, False)
        gather(0, True)

        def outer(g, _):
            for b in range(2):
                k = 2 * g + b
                nb = 1 - b
                gather(b, False)

                for p in range(2):
                    on = (g % 2) == p

                    @pl.when((k + 1 < kpt) & on)
                    def _(b=b, nb=nb, k=k, p=p):
                        ds1 = (nb + 2 * p) if b == 0 else 2 * (1 - p)
                        loads(nb, ds1, k + 1, False)
                        gather(nb, True)

                    @pl.when((k >= 2) & on)
                    def _(b=b, k=k, p=p):
                        scat(b, b + 2 * (1 - p), False)

                @plsc.parallel_loop(0, _C, unroll=2)
                def _(e):
                    for j in range(D // _L):
                        s = pl.ds(j * _L, _L)
                        msgs[b][e, s] = jnp.maximum(
                            rowss[b][e, s] + attrs[b][e, s], 0.0)

                for p in range(2):
                    on = (g % 2) == p

                    @pl.when(on)
                    def _(b=b, k=k, p=p):
                        scat(b, b + 2 * p, True)

                    @pl.when((k + 2 < kpt) & on)
                    def _(b=b, k=k, p=p):
                        loads(b, b + 2 * (1 - p), k + 2, True)
            return 0
        lax.fori_loop(0, kpt // 2, outer, 0)
        assert kpt % 4 == 2
        scat(0, 0, False)
        scat(1, 1, False)

        plsc.subcore_barrier()
        pltpu.sync_copy(acc.at[pl.ds(r0, rps), :],
                        out_hbm.at[cid, pl.ds(r0, rps), :])

    mesh = plsc.VectorSubcoreMesh(core_axis_name="c", subcore_axis_name="s")
    return pl.kernel(
        body,
        out_type=jax.ShapeDtypeStruct((_NC, npad, D), jnp.float32),
        mesh=mesh,
        scratch_types=[
            pltpu.VMEM((_C,), jnp.int32),
            pltpu.VMEM((_C,), jnp.int32),
            pltpu.VMEM((_C,), jnp.int32),
            pltpu.VMEM((_C,), jnp.int32),
            pltpu.VMEM((_C,), jnp.int32),
            pltpu.VMEM((_C,), jnp.int32),
            pltpu.VMEM((_C, D), jnp.float32),
            pltpu.VMEM((_C, D), jnp.float32),
            pltpu.VMEM((_C, D), jnp.float32),
            pltpu.VMEM((_C, D), jnp.float32),
            pltpu.VMEM((_C, D), jnp.float32),
            pltpu.VMEM((_C, D), jnp.float32),
            pltpu.VMEM_SHARED((npad, D), jnp.float32),
            pltpu.SemaphoreType.DMA,
            pltpu.SemaphoreType.DMA,
            pltpu.SemaphoreType.DMA,
            pltpu.SemaphoreType.DMA,
            pltpu.SemaphoreType.DMA,
            pltpu.SemaphoreType.DMA,
            pltpu.SemaphoreType.DMA,
            pltpu.SemaphoreType.DMA,
            pltpu.SemaphoreType.DMA,
            pltpu.SemaphoreType.DMA,
        ],
    )(x, src, dst, edge_attr, zrows)


def _mlp_body(x_ref, parts_ref, w1_ref, b1_ref, w2_ref, b2_ref, o_ref):
    h = x_ref[...] + parts_ref[0] + parts_ref[1]
    t = jnp.dot(h, w1_ref[...], preferred_element_type=jnp.float32,
                precision=jax.lax.Precision.HIGHEST) + b1_ref[...]
    t = jnp.maximum(t, 0.0)
    o_ref[...] = jnp.dot(t, w2_ref[...], preferred_element_type=jnp.float32,
                         precision=jax.lax.Precision.HIGHEST) + b2_ref[...]


def _tc_mlp(x, parts, W1, b1, W2, b2):
    N, D = x.shape
    R = 1000
    assert N % R == 0
    return pl.pallas_call(
        _mlp_body,
        grid=(N // R,),
        in_specs=[
            pl.BlockSpec((R, D), lambda i: (i, 0)),
            pl.BlockSpec((_NC, R, D), lambda i: (0, i, 0)),
            pl.BlockSpec((D, D), lambda i: (0, 0)),
            pl.BlockSpec((1, D), lambda i: (0, 0)),
            pl.BlockSpec((D, D), lambda i: (0, 0)),
            pl.BlockSpec((1, D), lambda i: (0, 0)),
        ],
        out_specs=pl.BlockSpec((R, D), lambda i: (i, 0)),
        out_shape=jax.ShapeDtypeStruct((N, D), jnp.float32),
    )(x, parts, W1, b1.reshape(1, D), W2, b2.reshape(1, D))


def kernel(x, edge_index, edge_attr, W1, b1, W2, b2):
    src = edge_index[0]
    dst = edge_index[1]
    N, D = x.shape
    npad = -(-N // (8 * _NS)) * (8 * _NS)
    zrows = jnp.zeros((npad // _NS, D), jnp.float32)
    parts = _sc_agg(x, src, dst, edge_attr, zrows)
    return _tc_mlp(x, parts, W1, b1, W2, b2)


# speedup vs baseline: 4.9538x; 1.0061x over previous
import jax
import jax.numpy as jnp
from jax import lax
from jax.experimental import pallas as pl
from jax.experimental.pallas import tpu as pltpu
from jax.experimental.pallas import tpu_sc as plsc

_NC = 2
_NS = 16
_L = 16
_C = 40


def _sc_agg(x, src, dst, edge_attr, zrows):
    N, D = x.shape
    E = src.shape[0]
    assert E % (_C * _NC * _NS) == 0 and D % _L == 0
    kpt = E // (_C * _NC * _NS)
    assert kpt % 2 == 0
    npad = -(-N // (8 * _NS)) * (8 * _NS)
    rps = npad // _NS

    def body(x_hbm, src_hbm, dst_hbm, attr_hbm, z_hbm, out_hbm,
             src0, src1, dst0, dst1, dst2, dst3, rows0, rows1,
             attr0, attr1, msg0, msg1, acc,
             l0, l1, g0, g1, s0, s1, d0, d1, d2, d3):
        cid = lax.axis_index("c")
        sid = lax.axis_index("s")
        wid = sid * _NC + cid
        srcs, dsts = (src0, src1), (dst0, dst1, dst2, dst3)
        rowss, attrs = (rows0, rows1), (attr0, attr1)
        msgs = (msg0, msg1)
        lsems, gsems, ssems = (l0, l1), (g0, g1), (s0, s1)
        dsems = (d0, d1, d2, d3)

        r0 = sid * rps
        pltpu.sync_copy(z_hbm, acc.at[pl.ds(r0, rps), :])
        plsc.subcore_barrier()

        def loads(b, ds, k, issue):
            base = (wid * kpt + k) * _C
            pairs = ((src_hbm.at[pl.ds(base, _C)], srcs[b], lsems[b]),
                     (attr_hbm.at[pl.ds(base, _C), :], attrs[b], lsems[b]),
                     (dst_hbm.at[pl.ds(base, _C)], dsts[ds], dsems[ds]))
            for s_ref, d_ref, sem in pairs:
                if issue:
                    pltpu.async_copy(s_ref, d_ref, sem)
                else:
                    pltpu.make_async_copy(s_ref, d_ref, sem).wait()

        def gather(b, issue):
            if issue:
                pltpu.async_copy(x_hbm.at[srcs[b]], rowss[b], gsems[b])
            else:
                pltpu.make_async_copy(x_hbm.at[srcs[b]], rowss[b], gsems[b]).wait()

        def scat(b, ds, issue):
            if issue:
                pltpu.async_copy(msgs[b], acc.at[dsts[ds]], ssems[b], add=True)
            else:
                pltpu.make_async_copy(msgs[b], acc.at[dsts[ds]], ssems[b]).wait()

        loads(0, 0, 0, True)
        loads(1, 1, 1, True)
        loads(0, 0, 0, False)
        gather(0, True)

        def outer(g, _):
            for b in range(2):
                k = 2 * g + b
                nb = 1 - b
                gather(b, False)

                for p in range(2):
                    on = (g % 2) == p

                    @pl.when((k + 1 < kpt) & on)
                    def _(b=b, nb=nb, k=k, p=p):
                        ds1 = (nb + 2 * p) if b == 0 else 2 * (1 - p)
                        loads(nb, ds1, k + 1, False)
                        gather(nb, True)

                    @pl.when((k >= 2) & on)
                    def _(b=b, k=k, p=p):
                        scat(b, b + 2 * (1 - p), False)

                @plsc.parallel_loop(0, _C, unroll=4)
                def _(e):
                    for j in range(D // _L):
                        s = pl.ds(j * _L, _L)
                        msgs[b][e, s] = jnp.maximum(
                            rowss[b][e, s] + attrs[b][e, s], 0.0)

                for p in range(2):
                    on = (g % 2) == p

                    @pl.when(on)
                    def _(b=b, k=k, p=p):
                        scat(b, b + 2 * p, True)

                    @pl.when((k + 2 < kpt) & on)
                    def _(b=b, k=k, p=p):
                        loads(b, b + 2 * (1 - p), k + 2, True)
            return 0
        lax.fori_loop(0, kpt // 2, outer, 0)
        assert kpt % 4 == 2
        scat(0, 0, False)
        scat(1, 1, False)

        plsc.subcore_barrier()
        pltpu.sync_copy(acc.at[pl.ds(r0, rps), :],
                        out_hbm.at[cid, pl.ds(r0, rps), :])

    mesh = plsc.VectorSubcoreMesh(core_axis_name="c", subcore_axis_name="s")
    return pl.kernel(
        body,
        out_type=jax.ShapeDtypeStruct((_NC, npad, D), jnp.float32),
        mesh=mesh,
        scratch_types=[
            pltpu.VMEM((_C,), jnp.int32),
            pltpu.VMEM((_C,), jnp.int32),
            pltpu.VMEM((_C,), jnp.int32),
            pltpu.VMEM((_C,), jnp.int32),
            pltpu.VMEM((_C,), jnp.int32),
            pltpu.VMEM((_C,), jnp.int32),
            pltpu.VMEM((_C, D), jnp.float32),
            pltpu.VMEM((_C, D), jnp.float32),
            pltpu.VMEM((_C, D), jnp.float32),
            pltpu.VMEM((_C, D), jnp.float32),
            pltpu.VMEM((_C, D), jnp.float32),
            pltpu.VMEM((_C, D), jnp.float32),
            pltpu.VMEM_SHARED((npad, D), jnp.float32),
            pltpu.SemaphoreType.DMA,
            pltpu.SemaphoreType.DMA,
            pltpu.SemaphoreType.DMA,
            pltpu.SemaphoreType.DMA,
            pltpu.SemaphoreType.DMA,
            pltpu.SemaphoreType.DMA,
            pltpu.SemaphoreType.DMA,
            pltpu.SemaphoreType.DMA,
            pltpu.SemaphoreType.DMA,
            pltpu.SemaphoreType.DMA,
        ],
    )(x, src, dst, edge_attr, zrows)


def _mlp_body(x_ref, parts_ref, w1_ref, b1_ref, w2_ref, b2_ref, o_ref):
    h = x_ref[...] + parts_ref[0] + parts_ref[1]
    t = jnp.dot(h, w1_ref[...], preferred_element_type=jnp.float32,
                precision=jax.lax.Precision.HIGHEST) + b1_ref[...]
    t = jnp.maximum(t, 0.0)
    o_ref[...] = jnp.dot(t, w2_ref[...], preferred_element_type=jnp.float32,
                         precision=jax.lax.Precision.HIGHEST) + b2_ref[...]


def _tc_mlp(x, parts, W1, b1, W2, b2):
    N, D = x.shape
    R = 1000
    assert N % R == 0
    return pl.pallas_call(
        _mlp_body,
        grid=(N // R,),
        in_specs=[
            pl.BlockSpec((R, D), lambda i: (i, 0)),
            pl.BlockSpec((_NC, R, D), lambda i: (0, i, 0)),
            pl.BlockSpec((D, D), lambda i: (0, 0)),
            pl.BlockSpec((1, D), lambda i: (0, 0)),
            pl.BlockSpec((D, D), lambda i: (0, 0)),
            pl.BlockSpec((1, D), lambda i: (0, 0)),
        ],
        out_specs=pl.BlockSpec((R, D), lambda i: (i, 0)),
        out_shape=jax.ShapeDtypeStruct((N, D), jnp.float32),
    )(x, parts, W1, b1.reshape(1, D), W2, b2.reshape(1, D))


def kernel(x, edge_index, edge_attr, W1, b1, W2, b2):
    src = edge_index[0]
    dst = edge_index[1]
    N, D = x.shape
    npad = -(-N // (8 * _NS)) * (8 * _NS)
    zrows = jnp.zeros((npad // _NS, D), jnp.float32)
    parts = _sc_agg(x, src, dst, edge_attr, zrows)
    return _tc_mlp(x, parts, W1, b1, W2, b2)
